# baseline (speedup 1.0000x reference)
import functools

import jax
import jax.numpy as jnp
from jax import lax
from jax.experimental import pallas as pl
from jax.experimental.pallas import tpu as pltpu
from jax.experimental.pallas import tpu_sc as plsc

_MARGIN = 0.05

_ROWS = 16384
_COLS = 1000
_E = 2000

_TC_ROWS = 9216
_SC_ROWS = _ROWS - _TC_ROWS

_NC = 2
_NS = 16
_L = 16
_NW = _NC * _NS
_RPW = _SC_ROWS // _NW
_R = 32
_NCHUNK = _RPW // _R
_NG = _E // _L

_BM = 512
_NBLK = _TC_ROWS // _BM

_mesh = plsc.VectorSubcoreMesh(core_axis_name="c", subcore_axis_name="s")


@functools.partial(
    pl.kernel,
    mesh=_mesh,
    out_type=jax.ShapeDtypeStruct((_NW, _L), jnp.float32),
    compiler_params=pltpu.CompilerParams(needs_layout_passes=False),
    scratch_types=[
        pltpu.VMEM((_E,), jnp.int32),
        pltpu.VMEM((_E,), jnp.int32),
        pltpu.VMEM((_E,), jnp.float32),
        pltpu.VMEM((_R, _COLS), jnp.float32),
        pltpu.VMEM((_R, _COLS), jnp.float32),
        pltpu.VMEM((_L,), jnp.float32),
        pltpu.SemaphoreType.DMA,
        pltpu.SemaphoreType.DMA,
    ],
)
def _hcl_sc(logits_hbm, cidx_hbm, pidx_hbm, w_hbm, out_hbm,
            cidx_v, pidx_v, w_v, buf0, buf1, out_v, sem0, sem1):
    wid = lax.axis_index("s") * _NC + lax.axis_index("c")
    row_base = _TC_ROWS + wid * _RPW

    pltpu.sync_copy(cidx_hbm, cidx_v)
    pltpu.sync_copy(pidx_hbm, pidx_v)
    pltpu.sync_copy(w_hbm, w_v)

    bufs = (buf0, buf1)
    sems = (sem0, sem1)
    copies = [None, None]
    copies[0] = pltpu.async_copy(
        logits_hbm.at[pl.ds(row_base, _R), :], buf0, sem0)

    zero = jnp.zeros((_L,), jnp.float32)
    neg_m = jnp.full((_L,), -_MARGIN, jnp.float32)
    accs = (zero, zero, zero, zero)

    for k in range(_NCHUNK):
        if k + 1 < _NCHUNK:
            nxt = (k + 1) % 2
            copies[nxt] = pltpu.async_copy(
                logits_hbm.at[pl.ds(row_base + (k + 1) * _R, _R), :],
                bufs[nxt], sems[nxt])
        copies[k % 2].wait()
        buf = bufs[k % 2]

        for half in range(_R // 16):

            def group_body(g, accs4, buf=buf, r0=half * 16):
                cvec = cidx_v[pl.ds(g * _L, _L)]
                pvec = pidx_v[pl.ds(g * _L, _L)]
                wvec = w_v[pl.ds(g * _L, _L)]
                a0, a1, a2, a3 = accs4
                for u in range(r0, r0 + 16):
                    ru = jnp.full((_L,), u, jnp.int32)
                    cu = plsc.load_gather(buf, [ru, cvec])
                    pu = plsc.load_gather(buf, [ru, pvec])
                    t = jnp.maximum(cu - pu, neg_m) * wvec
                    a0, a1, a2, a3 = a1, a2, a3, a0 + t
                return (a0, a1, a2, a3)

            accs = lax.fori_loop(0, _NG, group_body, accs)

    def wsum_body(g, s):
        return s + w_v[pl.ds(g * _L, _L)]
    wsum = lax.fori_loop(0, _NG, wsum_body, zero)

    total = (accs[0] + accs[1]) + (accs[2] + accs[3])
    total = total + (_MARGIN * _RPW) * wsum
    out_v[...] = total
    pltpu.sync_copy(out_v, out_hbm.at[wid])


def _hcl_tc_body(logits_ref, cidx_ref, pidx_ref, w_ref, out_ref,
                 g_ref, acc_ref):
    i = pl.program_id(0)

    @pl.when(i == 0)
    def _build_g():
        iota = lax.broadcasted_iota(jnp.int32, (_COLS, _E), 0)
        gm = (iota == cidx_ref[...]).astype(jnp.bfloat16)
        g_ref[...] = gm - (iota == pidx_ref[...]).astype(jnp.bfloat16)
        acc_ref[0, 0] = 0.0

    x = logits_ref[...]
    hi = x.astype(jnp.bfloat16)
    lo = (x - hi.astype(jnp.float32)).astype(jnp.bfloat16)
    gm = g_ref[...]
    d = (jnp.dot(hi, gm, preferred_element_type=jnp.float32)
         + jnp.dot(lo, gm, preferred_element_type=jnp.float32))
    t = jnp.maximum(d + _MARGIN, 0.0) * w_ref[...]
    acc_ref[0, 0] += jnp.sum(t)

    @pl.when(i == _NBLK - 1)
    def _emit():
        out_ref[0, 0] = acc_ref[0, 0]


_hcl_tc = pl.pallas_call(
    _hcl_tc_body,
    grid=(_NBLK,),
    in_specs=[
        pl.BlockSpec((_BM, _COLS), lambda i: (i, 0)),
        pl.BlockSpec((1, _E), lambda i: (0, 0)),
        pl.BlockSpec((1, _E), lambda i: (0, 0)),
        pl.BlockSpec((1, _E), lambda i: (0, 0)),
    ],
    out_specs=pl.BlockSpec(memory_space=pltpu.SMEM),
    out_shape=jax.ShapeDtypeStruct((1, 1), jnp.float32),
    scratch_shapes=[
        pltpu.VMEM((_COLS, _E), jnp.bfloat16),
        pltpu.SMEM((1, 1), jnp.float32),
    ],
)


def kernel(logits, edges_pc, weight):
    cidx = edges_pc[1].astype(jnp.int32)
    pidx = edges_pc[0].astype(jnp.int32)
    w32 = weight.astype(jnp.float32)
    sc_partials = _hcl_sc(logits, cidx, pidx, w32)
    tc_partial = _hcl_tc(logits, cidx[None, :], pidx[None, :], w32[None, :])
    total = jnp.sum(sc_partials) + tc_partial[0, 0]
    return total / (_ROWS * _E)


# speedup vs baseline: 1.2217x; 1.2217x over previous
import functools

import jax
import jax.numpy as jnp
from jax import lax
from jax.experimental import pallas as pl
from jax.experimental.pallas import tpu as pltpu
from jax.experimental.pallas import tpu_sc as plsc

_MARGIN = 0.05

_ROWS = 16384
_COLS = 1000
_E = 2000

_TC_ROWS = 9216
_SC_ROWS = _ROWS - _TC_ROWS

_NC = 2
_NS = 16
_L = 16
_NW = _NC * _NS
_RPW = _SC_ROWS // _NW
_R = 32
_NCHUNK = _RPW // _R
_NG = _E // _L

_BM = 512
_NBLK = _TC_ROWS // _BM

_mesh = plsc.VectorSubcoreMesh(core_axis_name="c", subcore_axis_name="s")


@functools.partial(
    pl.kernel,
    mesh=_mesh,
    out_type=jax.ShapeDtypeStruct((_NW, _L), jnp.float32),
    compiler_params=pltpu.CompilerParams(needs_layout_passes=False),
    scratch_types=[
        pltpu.VMEM((_E,), jnp.int32),
        pltpu.VMEM((_E,), jnp.int32),
        pltpu.VMEM((_E,), jnp.float32),
        pltpu.VMEM((_R, _COLS), jnp.float32),
        pltpu.VMEM((_R, _COLS), jnp.float32),
        pltpu.VMEM((_L,), jnp.float32),
        pltpu.SemaphoreType.DMA,
        pltpu.SemaphoreType.DMA,
    ],
)
def _hcl_sc(logits_hbm, cidx_hbm, pidx_hbm, w_hbm, out_hbm,
            cidx_v, pidx_v, w_v, buf0, buf1, out_v, sem0, sem1):
    wid = lax.axis_index("s") * _NC + lax.axis_index("c")
    row_base = _TC_ROWS + wid * _RPW

    pltpu.sync_copy(cidx_hbm, cidx_v)
    pltpu.sync_copy(pidx_hbm, pidx_v)
    pltpu.sync_copy(w_hbm, w_v)

    bufs = (buf0, buf1)
    sems = (sem0, sem1)
    copies = [None, None]
    copies[0] = pltpu.async_copy(
        logits_hbm.at[pl.ds(row_base, _R), :], buf0, sem0)

    zero = jnp.zeros((_L,), jnp.float32)
    neg_m = jnp.full((_L,), -_MARGIN, jnp.float32)
    accs = (zero, zero, zero, zero)

    for k in range(_NCHUNK):
        if k + 1 < _NCHUNK:
            nxt = (k + 1) % 2
            copies[nxt] = pltpu.async_copy(
                logits_hbm.at[pl.ds(row_base + (k + 1) * _R, _R), :],
                bufs[nxt], sems[nxt])
        copies[k % 2].wait()
        buf = bufs[k % 2]

        for half in range(_R // 16):

            def group_body(g, accs4, buf=buf, r0=half * 16):
                cvec = cidx_v[pl.ds(g * _L, _L)]
                pvec = pidx_v[pl.ds(g * _L, _L)]
                wvec = w_v[pl.ds(g * _L, _L)]
                a0, a1, a2, a3 = accs4
                for u in range(r0, r0 + 16):
                    ru = jnp.full((_L,), u, jnp.int32)
                    cu = plsc.load_gather(buf, [ru, cvec])
                    pu = plsc.load_gather(buf, [ru, pvec])
                    t = jnp.maximum(cu - pu, neg_m) * wvec
                    a0, a1, a2, a3 = a1, a2, a3, a0 + t
                return (a0, a1, a2, a3)

            accs = lax.fori_loop(0, _NG, group_body, accs)

    def wsum_body(g, s):
        return s + w_v[pl.ds(g * _L, _L)]
    wsum = lax.fori_loop(0, _NG, wsum_body, zero)

    total = (accs[0] + accs[1]) + (accs[2] + accs[3])
    total = total + (_MARGIN * _RPW) * wsum
    out_v[...] = total
    pltpu.sync_copy(out_v, out_hbm.at[wid])


def _hcl_tc_body(logits_ref, cidx_ref, pidx_ref, w_ref, out_ref,
                 g_ref, acc_ref):
    i = pl.program_id(0)

    @pl.when(i == 0)
    def _build_g():
        iota = lax.broadcasted_iota(jnp.int32, (_COLS, _E), 0)
        gm = (iota == cidx_ref[...]).astype(jnp.bfloat16)
        g_ref[...] = gm - (iota == pidx_ref[...]).astype(jnp.bfloat16)
        acc_ref[0, 0] = 0.0

    x = logits_ref[...].astype(jnp.bfloat16)
    gm = g_ref[...]
    d = jnp.dot(x, gm, preferred_element_type=jnp.float32)
    t = jnp.maximum(d + _MARGIN, 0.0) * w_ref[...]
    acc_ref[0, 0] += jnp.sum(t)

    @pl.when(i == _NBLK - 1)
    def _emit():
        out_ref[0, 0] = acc_ref[0, 0]


_hcl_tc = pl.pallas_call(
    _hcl_tc_body,
    grid=(_NBLK,),
    in_specs=[
        pl.BlockSpec((_BM, _COLS), lambda i: (i, 0)),
        pl.BlockSpec((1, _E), lambda i: (0, 0)),
        pl.BlockSpec((1, _E), lambda i: (0, 0)),
        pl.BlockSpec((1, _E), lambda i: (0, 0)),
    ],
    out_specs=pl.BlockSpec(memory_space=pltpu.SMEM),
    out_shape=jax.ShapeDtypeStruct((1, 1), jnp.float32),
    scratch_shapes=[
        pltpu.VMEM((_COLS, _E), jnp.bfloat16),
        pltpu.SMEM((1, 1), jnp.float32),
    ],
)


def kernel(logits, edges_pc, weight):
    cidx = edges_pc[1].astype(jnp.int32)
    pidx = edges_pc[0].astype(jnp.int32)
    w32 = weight.astype(jnp.float32)
    sc_partials = _hcl_sc(logits, cidx, pidx, w32)
    tc_partial = _hcl_tc(logits, cidx[None, :], pidx[None, :], w32[None, :])
    total = jnp.sum(sc_partials) + tc_partial[0, 0]
    return total / (_ROWS * _E)


# speedup vs baseline: 1.8398x; 1.5060x over previous
import functools

import jax
import jax.numpy as jnp
from jax import lax
from jax.experimental import pallas as pl
from jax.experimental.pallas import tpu as pltpu
from jax.experimental.pallas import tpu_sc as plsc

_MARGIN = 0.05

_ROWS = 16384
_COLS = 1000
_E = 2000
_EPAD = 2048

_TC_ROWS = 9216
_SC_ROWS = _ROWS - _TC_ROWS

_NC = 2
_NS = 16
_L = 16
_NW = _NC * _NS
_EPW = _EPAD // _NW
_NV = _SC_ROWS // _L
_UNROLL = 4

_BM = 512
_NBLK = _TC_ROWS // _BM

_mesh = plsc.VectorSubcoreMesh(core_axis_name="c", subcore_axis_name="s")


@functools.partial(
    pl.kernel,
    mesh=_mesh,
    out_type=jax.ShapeDtypeStruct((_NW, _L), jnp.float32),
    compiler_params=pltpu.CompilerParams(needs_layout_passes=False),
    scratch_types=[
        pltpu.VMEM((_EPAD,), jnp.int32),
        pltpu.VMEM((_EPAD,), jnp.int32),
        pltpu.VMEM((_EPAD,), jnp.float32),
        pltpu.VMEM((_NV * _L,), jnp.float32),
        pltpu.VMEM((_NV * _L,), jnp.float32),
        pltpu.VMEM((_NV * _L,), jnp.float32),
        pltpu.VMEM((_NV * _L,), jnp.float32),
        pltpu.VMEM((_L,), jnp.float32),
        pltpu.SemaphoreType.DMA,
        pltpu.SemaphoreType.DMA,
    ],
)
def _hcl_sc(logits_t_hbm, cidx_hbm, pidx_hbm, w_hbm, out_hbm,
            cidx_v, pidx_v, w_v, cb0, cb1, pb0, pb1, out_v, sem0, sem1):
    wid = lax.axis_index("s") * _NC + lax.axis_index("c")
    ebase = wid * _EPW

    pltpu.sync_copy(cidx_hbm, cidx_v)
    pltpu.sync_copy(pidx_hbm, pidx_v)
    pltpu.sync_copy(w_hbm, w_v)

    cvecs = [cidx_v[pl.ds(ebase + i * _L, _L)] for i in range(_EPW // _L)]
    pvecs = [pidx_v[pl.ds(ebase + i * _L, _L)] for i in range(_EPW // _L)]
    wvecs = [w_v[pl.ds(ebase + i * _L, _L)] for i in range(_EPW // _L)]

    def col(vecs, j):
        return vecs[j // _L][j % _L]

    cbufs = (cb0, cb1)
    pbufs = (pb0, pb1)
    sems = (sem0, sem1)
    copies = [None, None]

    def start(j):
        par = j % 2
        cc = pltpu.async_copy(
            logits_t_hbm.at[col(cvecs, j), pl.ds(_TC_ROWS, _SC_ROWS)],
            cbufs[par], sems[par])
        pc = pltpu.async_copy(
            logits_t_hbm.at[col(pvecs, j), pl.ds(_TC_ROWS, _SC_ROWS)],
            pbufs[par], sems[par])
        copies[par] = (cc, pc)

    start(0)

    zero = jnp.zeros((_L,), jnp.float32)
    neg_m = jnp.full((_L,), -_MARGIN, jnp.float32)
    accs = [zero] * _UNROLL

    for j in range(_EPW):
        if j + 1 < _EPW:
            start(j + 1)
        par = j % 2
        copies[par][0].wait()
        copies[par][1].wait()
        cb, pb = cbufs[par], pbufs[par]
        wj = jnp.full((_L,), col(wvecs, j), jnp.float32)

        def vec_body(i, accs4, cb=cb, pb=pb, wj=wj):
            o = i * (_UNROLL * _L)
            out = list(accs4)
            for u in range(_UNROLL):
                cv = cb[pl.ds(o + u * _L, _L)]
                pv = pb[pl.ds(o + u * _L, _L)]
                out[u] = out[u] + jnp.maximum(cv - pv, neg_m) * wj
            return tuple(out)

        accs = list(lax.fori_loop(0, _NV // _UNROLL, vec_body, tuple(accs)))

    wsum = zero
    for i in range(_EPW // _L):
        wsum = wsum + wvecs[i]

    total = (accs[0] + accs[1]) + (accs[2] + accs[3])
    total = total + (_MARGIN * _SC_ROWS) * wsum
    out_v[...] = total
    pltpu.sync_copy(out_v, out_hbm.at[wid])


def _hcl_tc_body(logits_t_ref, cidx_ref, pidx_ref, w_ref, out_ref,
                 g_ref, acc_ref):
    i = pl.program_id(0)

    @pl.when(i == 0)
    def _build_g():
        iota = lax.broadcasted_iota(jnp.int32, (_E, _COLS), 1)
        gm = (iota == cidx_ref[...]).astype(jnp.bfloat16)
        g_ref[...] = gm - (iota == pidx_ref[...]).astype(jnp.bfloat16)
        acc_ref[0, 0] = 0.0

    x = logits_t_ref[...].astype(jnp.bfloat16)
    d = jnp.dot(g_ref[...], x, preferred_element_type=jnp.float32)
    t = jnp.maximum(d + _MARGIN, 0.0) * w_ref[...]
    acc_ref[0, 0] += jnp.sum(t)

    @pl.when(i == _NBLK - 1)
    def _emit():
        out_ref[0, 0] = acc_ref[0, 0]


_hcl_tc = pl.pallas_call(
    _hcl_tc_body,
    grid=(_NBLK,),
    in_specs=[
        pl.BlockSpec((_COLS, _BM), lambda i: (0, i)),
        pl.BlockSpec((_E, 1), lambda i: (0, 0)),
        pl.BlockSpec((_E, 1), lambda i: (0, 0)),
        pl.BlockSpec((_E, 1), lambda i: (0, 0)),
    ],
    out_specs=pl.BlockSpec(memory_space=pltpu.SMEM),
    out_shape=jax.ShapeDtypeStruct((1, 1), jnp.float32),
    scratch_shapes=[
        pltpu.VMEM((_E, _COLS), jnp.bfloat16),
        pltpu.SMEM((1, 1), jnp.float32),
    ],
)


def kernel(logits, edges_pc, weight):
    cidx = edges_pc[1].astype(jnp.int32)
    pidx = edges_pc[0].astype(jnp.int32)
    w32 = weight.astype(jnp.float32)
    logits_t = logits.T
    cpad = jnp.pad(cidx, (0, _EPAD - _E))
    ppad = jnp.pad(pidx, (0, _EPAD - _E))
    wpad = jnp.pad(w32, (0, _EPAD - _E))
    sc_partials = _hcl_sc(logits_t, cpad, ppad, wpad)
    tc_partial = _hcl_tc(logits_t, cidx[:, None], pidx[:, None],
                         w32[:, None])
    total = jnp.sum(sc_partials) + tc_partial[0, 0]
    return total / (_ROWS * _E)


# speedup vs baseline: 1.8561x; 1.0088x over previous
import functools

import jax
import jax.numpy as jnp
from jax import lax
from jax.experimental import pallas as pl
from jax.experimental.pallas import tpu as pltpu
from jax.experimental.pallas import tpu_sc as plsc

_MARGIN = 0.05

_ROWS = 16384
_COLS = 1000
_E = 2000
_EPAD = 2048

_TC_ROWS = 9728
_SC_ROWS = _ROWS - _TC_ROWS

_NC = 2
_NS = 16
_L = 16
_NW = _NC * _NS
_EPW = _EPAD // _NW
_NV = _SC_ROWS // _L
_UNROLL = 8

_BM = 512
_NBLK = _TC_ROWS // _BM

_mesh = plsc.VectorSubcoreMesh(core_axis_name="c", subcore_axis_name="s")


@functools.partial(
    pl.kernel,
    mesh=_mesh,
    out_type=jax.ShapeDtypeStruct((_NW, _L), jnp.float32),
    compiler_params=pltpu.CompilerParams(needs_layout_passes=False),
    scratch_types=[
        pltpu.VMEM((_EPAD,), jnp.int32),
        pltpu.VMEM((_EPAD,), jnp.int32),
        pltpu.VMEM((_EPAD,), jnp.float32),
        pltpu.VMEM((_NV * _L,), jnp.float32),
        pltpu.VMEM((_NV * _L,), jnp.float32),
        pltpu.VMEM((_NV * _L,), jnp.float32),
        pltpu.VMEM((_NV * _L,), jnp.float32),
        pltpu.VMEM((_L,), jnp.float32),
        pltpu.SemaphoreType.DMA,
        pltpu.SemaphoreType.DMA,
    ],
)
def _hcl_sc(logits_t_hbm, cidx_hbm, pidx_hbm, w_hbm, out_hbm,
            cidx_v, pidx_v, w_v, cb0, cb1, pb0, pb1, out_v, sem0, sem1):
    wid = lax.axis_index("s") * _NC + lax.axis_index("c")
    ebase = wid * _EPW

    pltpu.sync_copy(cidx_hbm, cidx_v)
    pltpu.sync_copy(pidx_hbm, pidx_v)
    pltpu.sync_copy(w_hbm, w_v)

    cvecs = [cidx_v[pl.ds(ebase + i * _L, _L)] for i in range(_EPW // _L)]
    pvecs = [pidx_v[pl.ds(ebase + i * _L, _L)] for i in range(_EPW // _L)]
    wvecs = [w_v[pl.ds(ebase + i * _L, _L)] for i in range(_EPW // _L)]

    def col(vecs, j):
        return vecs[j // _L][j % _L]

    cbufs = (cb0, cb1)
    pbufs = (pb0, pb1)
    sems = (sem0, sem1)
    copies = [None, None]

    def start(j):
        par = j % 2
        cc = pltpu.async_copy(
            logits_t_hbm.at[col(cvecs, j), pl.ds(_TC_ROWS, _SC_ROWS)],
            cbufs[par], sems[par])
        pc = pltpu.async_copy(
            logits_t_hbm.at[col(pvecs, j), pl.ds(_TC_ROWS, _SC_ROWS)],
            pbufs[par], sems[par])
        copies[par] = (cc, pc)

    start(0)

    zero = jnp.zeros((_L,), jnp.float32)
    neg_m = jnp.full((_L,), -_MARGIN, jnp.float32)
    accs = [zero] * _UNROLL

    for j in range(_EPW):
        if j + 1 < _EPW:
            start(j + 1)
        par = j % 2
        copies[par][0].wait()
        copies[par][1].wait()
        cb, pb = cbufs[par], pbufs[par]
        wj = jnp.full((_L,), col(wvecs, j), jnp.float32)

        def vec_body(i, accs4, cb=cb, pb=pb, wj=wj):
            o = i * (_UNROLL * _L)
            out = list(accs4)
            for u in range(_UNROLL):
                cv = cb[pl.ds(o + u * _L, _L)]
                pv = pb[pl.ds(o + u * _L, _L)]
                out[u] = out[u] + jnp.maximum(cv - pv, neg_m) * wj
            return tuple(out)

        accs = list(lax.fori_loop(0, _NV // _UNROLL, vec_body, tuple(accs)))

    wsum = zero
    for i in range(_EPW // _L):
        wsum = wsum + wvecs[i]

    while len(accs) > 1:
        accs = [a + b for a, b in zip(accs[::2], accs[1::2])]
    total = accs[0] + (_MARGIN * _SC_ROWS) * wsum
    out_v[...] = total
    pltpu.sync_copy(out_v, out_hbm.at[wid])


def _hcl_tc_body(logits_t_ref, cidx_ref, pidx_ref, w_ref, out_ref,
                 g_ref, acc_ref):
    i = pl.program_id(0)

    @pl.when(i == 0)
    def _build_g():
        iota = lax.broadcasted_iota(jnp.int32, (_E, _COLS), 1)
        gm = (iota == cidx_ref[...]).astype(jnp.bfloat16)
        g_ref[...] = gm - (iota == pidx_ref[...]).astype(jnp.bfloat16)
        acc_ref[0, 0] = 0.0

    x = logits_t_ref[...].astype(jnp.bfloat16)
    d = jnp.dot(g_ref[...], x, preferred_element_type=jnp.float32)
    t = jnp.maximum(d + _MARGIN, 0.0) * w_ref[...]
    acc_ref[0, 0] += jnp.sum(t)

    @pl.when(i == _NBLK - 1)
    def _emit():
        out_ref[0, 0] = acc_ref[0, 0]


_hcl_tc = pl.pallas_call(
    _hcl_tc_body,
    grid=(_NBLK,),
    in_specs=[
        pl.BlockSpec((_COLS, _BM), lambda i: (0, i)),
        pl.BlockSpec((_E, 1), lambda i: (0, 0)),
        pl.BlockSpec((_E, 1), lambda i: (0, 0)),
        pl.BlockSpec((_E, 1), lambda i: (0, 0)),
    ],
    out_specs=pl.BlockSpec(memory_space=pltpu.SMEM),
    out_shape=jax.ShapeDtypeStruct((1, 1), jnp.float32),
    scratch_shapes=[
        pltpu.VMEM((_E, _COLS), jnp.bfloat16),
        pltpu.SMEM((1, 1), jnp.float32),
    ],
)


def kernel(logits, edges_pc, weight):
    cidx = edges_pc[1].astype(jnp.int32)
    pidx = edges_pc[0].astype(jnp.int32)
    w32 = weight.astype(jnp.float32)
    logits_t = logits.T
    cpad = jnp.pad(cidx, (0, _EPAD - _E))
    ppad = jnp.pad(pidx, (0, _EPAD - _E))
    wpad = jnp.pad(w32, (0, _EPAD - _E))
    sc_partials = _hcl_sc(logits_t, cpad, ppad, wpad)
    tc_partial = _hcl_tc(logits_t, cidx[:, None], pidx[:, None],
                         w32[:, None])
    total = jnp.sum(sc_partials) + tc_partial[0, 0]
    return total / (_ROWS * _E)


# speedup vs baseline: 2.0392x; 1.0987x over previous
import functools

import jax
import jax.numpy as jnp
from jax import lax
from jax.experimental import pallas as pl
from jax.experimental.pallas import tpu as pltpu
from jax.experimental.pallas import tpu_sc as plsc

_MARGIN = 0.05

_ROWS = 16384
_COLS = 1000
_E = 2000
_EPAD = 2048

_TC_ROWS = 10240
_SC_ROWS = _ROWS - _TC_ROWS

_NC = 2
_NS = 16
_L = 16
_NW = _NC * _NS
_EPW = _EPAD // _NW
_NV = _SC_ROWS // _L
_UNROLL = 8

_BM = 512
_NBLK = _TC_ROWS // _BM

_mesh = plsc.VectorSubcoreMesh(core_axis_name="c", subcore_axis_name="s")


@functools.partial(
    pl.kernel,
    mesh=_mesh,
    out_type=jax.ShapeDtypeStruct((_NW, _L), jnp.float32),
    compiler_params=pltpu.CompilerParams(needs_layout_passes=False),
    scratch_types=[
        pltpu.VMEM((_EPAD,), jnp.int32),
        pltpu.VMEM((_EPAD,), jnp.int32),
        pltpu.VMEM((_EPAD,), jnp.float32),
        pltpu.VMEM((_NV * _L,), jnp.float32),
        pltpu.VMEM((_NV * _L,), jnp.float32),
        pltpu.VMEM((_NV * _L,), jnp.float32),
        pltpu.VMEM((_NV * _L,), jnp.float32),
        pltpu.VMEM((_NV * _L,), jnp.float32),
        pltpu.VMEM((_NV * _L,), jnp.float32),
        pltpu.VMEM((_L,), jnp.float32),
        pltpu.SemaphoreType.DMA,
        pltpu.SemaphoreType.DMA,
        pltpu.SemaphoreType.DMA,
    ],
)
def _hcl_sc(logits_t_hbm, cidx_hbm, pidx_hbm, w_hbm, out_hbm,
            cidx_v, pidx_v, w_v, cb0, cb1, cb2, pb0, pb1, pb2, out_v,
            sem0, sem1, sem2):
    wid = lax.axis_index("s") * _NC + lax.axis_index("c")
    ebase = wid * _EPW

    pltpu.sync_copy(cidx_hbm, cidx_v)
    pltpu.sync_copy(pidx_hbm, pidx_v)
    pltpu.sync_copy(w_hbm, w_v)

    cvecs = [cidx_v[pl.ds(ebase + i * _L, _L)] for i in range(_EPW // _L)]
    pvecs = [pidx_v[pl.ds(ebase + i * _L, _L)] for i in range(_EPW // _L)]
    wvecs = [w_v[pl.ds(ebase + i * _L, _L)] for i in range(_EPW // _L)]

    def col(vecs, j):
        return vecs[j // _L][j % _L]

    cbufs = (cb0, cb1, cb2)
    pbufs = (pb0, pb1, pb2)
    sems = (sem0, sem1, sem2)
    copies = [None, None, None]

    def start(j):
        par = j % 3
        cc = pltpu.async_copy(
            logits_t_hbm.at[col(cvecs, j), pl.ds(_TC_ROWS, _SC_ROWS)],
            cbufs[par], sems[par])
        pc = pltpu.async_copy(
            logits_t_hbm.at[col(pvecs, j), pl.ds(_TC_ROWS, _SC_ROWS)],
            pbufs[par], sems[par])
        copies[par] = (cc, pc)

    start(0)
    start(1)

    zero = jnp.zeros((_L,), jnp.float32)
    neg_m = jnp.full((_L,), -_MARGIN, jnp.float32)
    accs = [zero] * _UNROLL

    for j in range(_EPW):
        if j + 2 < _EPW:
            start(j + 2)
        par = j % 3
        copies[par][0].wait()
        copies[par][1].wait()
        cb, pb = cbufs[par], pbufs[par]
        wj = jnp.full((_L,), col(wvecs, j), jnp.float32)

        def vec_body(i, accs4, cb=cb, pb=pb, wj=wj):
            o = i * (_UNROLL * _L)
            out = list(accs4)
            for u in range(_UNROLL):
                cv = cb[pl.ds(o + u * _L, _L)]
                pv = pb[pl.ds(o + u * _L, _L)]
                out[u] = out[u] + jnp.maximum(cv - pv, neg_m) * wj
            return tuple(out)

        accs = list(lax.fori_loop(0, _NV // _UNROLL, vec_body, tuple(accs)))

    wsum = zero
    for i in range(_EPW // _L):
        wsum = wsum + wvecs[i]

    while len(accs) > 1:
        accs = [a + b for a, b in zip(accs[::2], accs[1::2])]
    total = accs[0] + (_MARGIN * _SC_ROWS) * wsum
    out_v[...] = total
    pltpu.sync_copy(out_v, out_hbm.at[wid])


def _hcl_tc_body(logits_t_ref, cidx_ref, pidx_ref, w_ref, out_ref,
                 g_ref, acc_ref):
    i = pl.program_id(0)

    @pl.when(i == 0)
    def _build_g():
        iota = lax.broadcasted_iota(jnp.int32, (_E, _COLS), 1)
        gm = (iota == cidx_ref[...]).astype(jnp.bfloat16)
        g_ref[...] = gm - (iota == pidx_ref[...]).astype(jnp.bfloat16)
        acc_ref[0, 0] = 0.0

    x = logits_t_ref[...].astype(jnp.bfloat16)
    d = jnp.dot(g_ref[...], x, preferred_element_type=jnp.float32)
    t = jnp.maximum(d + _MARGIN, 0.0) * w_ref[...]
    acc_ref[0, 0] += jnp.sum(t)

    @pl.when(i == _NBLK - 1)
    def _emit():
        out_ref[0, 0] = acc_ref[0, 0]


_hcl_tc = pl.pallas_call(
    _hcl_tc_body,
    grid=(_NBLK,),
    in_specs=[
        pl.BlockSpec((_COLS, _BM), lambda i: (0, i)),
        pl.BlockSpec((_E, 1), lambda i: (0, 0)),
        pl.BlockSpec((_E, 1), lambda i: (0, 0)),
        pl.BlockSpec((_E, 1), lambda i: (0, 0)),
    ],
    out_specs=pl.BlockSpec(memory_space=pltpu.SMEM),
    out_shape=jax.ShapeDtypeStruct((1, 1), jnp.float32),
    scratch_shapes=[
        pltpu.VMEM((_E, _COLS), jnp.bfloat16),
        pltpu.SMEM((1, 1), jnp.float32),
    ],
)


def kernel(logits, edges_pc, weight):
    cidx = edges_pc[1].astype(jnp.int32)
    pidx = edges_pc[0].astype(jnp.int32)
    w32 = weight.astype(jnp.float32)
    logits_t = logits.T
    cpad = jnp.pad(cidx, (0, _EPAD - _E))
    ppad = jnp.pad(pidx, (0, _EPAD - _E))
    wpad = jnp.pad(w32, (0, _EPAD - _E))
    sc_partials = _hcl_sc(logits_t, cpad, ppad, wpad)
    tc_partial = _hcl_tc(logits_t, cidx[:, None], pidx[:, None],
                         w32[:, None])
    total = jnp.sum(sc_partials) + tc_partial[0, 0]
    return total / (_ROWS * _E)
